# baseline (speedup 1.0000x reference)
import functools

import jax
import jax.numpy as jnp
from jax import lax
from jax.experimental import pallas as pl
from jax.experimental.pallas import tpu as pltpu
from jax.experimental.pallas import tpu_sc as plsc

BATCH = 4096
HIST = 200
DIM = 32
N = BATCH * HIST
NC = 2
NS = 16
NW = NC * NS
PER_W = N // NW
CHUNK = 128
NCHUNK = PER_W // CHUNK
ROW_UNROLL = 8
OUT_W = 2 * DIM


def _sc_body(idx_hbm, wr_hbm, wi_hbm, out_hbm, idx_v, r_v, i_v, o_v,
             sem_r, sem_i):
    wid = lax.axis_index("s") * NC + lax.axis_index("c")
    two_iota = lax.iota(jnp.int32, 16) * 2

    def body(g, carry):
        base = wid * PER_W + g * CHUNK
        pltpu.sync_copy(idx_hbm.at[pl.ds(base, CHUNK)], idx_v)
        cr = pltpu.async_copy(wr_hbm.at[idx_v], r_v, sem_r)
        ci = pltpu.async_copy(wi_hbm.at[idx_v], i_v, sem_i)
        cr.wait()
        ci.wait()

        def rows(rb, carry2):
            for u in range(ROW_UNROLL):
                row = rb * ROW_UNROLL + u
                o_base = row * OUT_W
                for half in range(2):
                    rv = r_v[row, pl.ds(16 * half, 16)]
                    iv = i_v[row, pl.ds(16 * half, 16)]
                    col = o_base + 32 * half + two_iota
                    plsc.store_scatter(o_v, [col], rv)
                    plsc.store_scatter(o_v, [col + 1], iv)
            return carry2

        lax.fori_loop(0, CHUNK // ROW_UNROLL, rows, 0)
        pltpu.sync_copy(o_v, out_hbm.at[pl.ds(base * OUT_W, CHUNK * OUT_W)])
        return carry

    lax.fori_loop(0, NCHUNK, body, 0)


@jax.jit
def _complex_embed(ids, W_real, W_imag):
    run = pl.kernel(
        _sc_body,
        out_type=jax.ShapeDtypeStruct((N * OUT_W,), jnp.float32),
        mesh=plsc.VectorSubcoreMesh(core_axis_name="c", subcore_axis_name="s"),
        compiler_params=pltpu.CompilerParams(
            use_tc_tiling_on_sc=False, needs_layout_passes=False),
        scratch_types=[
            pltpu.VMEM((CHUNK,), jnp.int32),
            pltpu.VMEM((CHUNK, DIM), jnp.float32),
            pltpu.VMEM((CHUNK, DIM), jnp.float32),
            pltpu.VMEM((CHUNK * OUT_W,), jnp.float32),
            pltpu.SemaphoreType.DMA,
            pltpu.SemaphoreType.DMA,
        ],
    )
    return run(ids, W_real, W_imag)


def kernel(token_ids, W_real, W_imag):
    ids = token_ids.reshape(N).astype(jnp.int32)
    out = _complex_embed(ids, W_real, W_imag)
    return out.reshape(BATCH, HIST, DIM, 2)
